# baseline (speedup 1.0000x reference)
import jax
import jax.numpy as jnp
from jax import lax
from jax.experimental import pallas as pl
from jax.experimental.pallas import tpu as pltpu

_NEXP = 64
_TOPK = 2
_CHUNK = 512
_NBUF = 4


def _gate(x_hbm, w_ref, wout_ref, iout_ref, buf, sems):
    ntok = x_hbm.shape[0]
    nchunk = ntok // _CHUNK
    w = w_ref[...]

    def start(i, slot):
        pltpu.make_async_copy(
            x_hbm.at[pl.ds(i * _CHUNK, _CHUNK), :],
            buf.at[slot],
            sems.at[slot],
        ).start()

    def wait(i, slot):
        pltpu.make_async_copy(
            x_hbm.at[pl.ds(i * _CHUNK, _CHUNK), :],
            buf.at[slot],
            sems.at[slot],
        ).wait()

    for p in range(_NBUF):
        start(p, p)

    def body(i, carry):
        slot = lax.rem(i, _NBUF)
        wait(i, slot)
        x = buf[slot]
        logits = lax.dot_general(
            x, w, (((1,), (1,)), ((), ())),
            preferred_element_type=jnp.float32)

        ids = lax.broadcasted_iota(jnp.int32, logits.shape, 1)
        m1 = jnp.max(logits, axis=1, keepdims=True)
        denom = jnp.sum(jnp.exp(logits - m1), axis=1, keepdims=True)
        big = jnp.int32(_NEXP)
        i1 = jnp.min(jnp.where(logits == m1, ids, big), axis=1, keepdims=True)
        masked = jnp.where(ids == i1, -jnp.inf, logits)
        m2 = jnp.max(masked, axis=1, keepdims=True)
        i2 = jnp.min(jnp.where(masked == m2, ids, big), axis=1, keepdims=True)

        w1 = jnp.exp(m1 - m1) / denom
        w2 = jnp.exp(m2 - m1) / denom

        slot2 = lax.broadcasted_iota(jnp.int32, (_CHUNK, _TOPK), 1)
        wout_ref[pl.ds(i * _CHUNK, _CHUNK), :] = jnp.where(slot2 == 0, w1, w2)
        iout_ref[pl.ds(i * _CHUNK, _CHUNK), :] = jnp.where(slot2 == 0, i1, i2)

        nxt = i + _NBUF

        @pl.when(nxt < nchunk)
        def _():
            start(nxt, slot)

        return carry

    lax.fori_loop(0, nchunk, body, jnp.int32(0))


def kernel(x, W):
    ntok, dim = x.shape
    weights, indices = pl.pallas_call(
        _gate,
        in_specs=[
            pl.BlockSpec(memory_space=pl.ANY),
            pl.BlockSpec(memory_space=pltpu.VMEM),
        ],
        out_specs=[
            pl.BlockSpec(memory_space=pltpu.VMEM),
            pl.BlockSpec(memory_space=pltpu.VMEM),
        ],
        out_shape=[
            jax.ShapeDtypeStruct((ntok, _TOPK), jnp.float32),
            jax.ShapeDtypeStruct((ntok, _TOPK), jnp.int32),
        ],
        scratch_shapes=[
            pltpu.VMEM((_NBUF, _CHUNK, dim), jnp.float32),
            pltpu.SemaphoreType.DMA((_NBUF,)),
        ],
    )(x, W)
    return (weights, indices)
